# Initial kernel scaffold; baseline (speedup 1.0000x reference)
import jax
import jax.numpy as jnp
from jax.experimental import pallas as pl


def kernel(x, edge_index, edge_type, W_rel, W_0, W_g, b_g, W_s, b_s):
    raise NotImplementedError("write your pallas kernel here")


# speedup vs baseline: 987.8302x; 987.8302x over previous
import jax
import jax.numpy as jnp
from jax import lax
from jax.experimental import pallas as pl
from jax.experimental.pallas import tpu as pltpu
from jax.experimental.pallas import tpu_sc as plsc

_B, _N, _E, _D, _R = 32, 512, 8192, 256, 8
_NG, _NS = 10000, 2000
_L = 16


def _sc_hist_body(ei_hbm, et_hbm, deg_hbm, cnt_hbm,
                  src_v, dst_v, et_v, histd_v, histc_v):
    c = lax.axis_index("c")
    s = lax.axis_index("s")
    w = s * 2 + c

    pltpu.sync_copy(ei_hbm.at[w, 0], src_v)
    pltpu.sync_copy(ei_hbm.at[w, 1], dst_v)
    pltpu.sync_copy(et_hbm.at[w], et_v)

    zeros = jnp.zeros((_L,), jnp.float32)

    def zero_body(i, carry):
        histd_v[pl.ds(i * _L, _L)] = zeros
        histc_v[pl.ds(i * _L, _L)] = zeros
        return carry

    lax.fori_loop(0, (_R * _N) // _L, zero_body, 0)

    ones = jnp.ones((_L,), jnp.float32)

    def edge_body(i, carry):
        off = i * _L
        sv = src_v[pl.ds(off, _L)]
        dv = dst_v[pl.ds(off, _L)]
        tv = et_v[pl.ds(off, _L)]
        base = tv * _N
        plsc.addupdate_scatter(histd_v, [base + dv], ones)
        plsc.addupdate_scatter(histc_v, [base + sv], ones, mask=dv == 0)
        return carry

    lax.fori_loop(0, _E // _L, edge_body, 0)

    pltpu.sync_copy(histd_v, deg_hbm.at[w])
    pltpu.sync_copy(histc_v, cnt_hbm.at[w])


def _sc_histograms(edge_index, edge_type):
    mesh = plsc.VectorSubcoreMesh(core_axis_name="c", subcore_axis_name="s")
    fn = pl.kernel(
        _sc_hist_body,
        out_type=[
            jax.ShapeDtypeStruct((_B, _R * _N), jnp.float32),
            jax.ShapeDtypeStruct((_B, _R * _N), jnp.float32),
        ],
        mesh=mesh,
        compiler_params=pltpu.CompilerParams(needs_layout_passes=False),
        scratch_types=[
            pltpu.VMEM((_E,), jnp.int32),
            pltpu.VMEM((_E,), jnp.int32),
            pltpu.VMEM((_E,), jnp.int32),
            pltpu.VMEM((_R * _N,), jnp.float32),
            pltpu.VMEM((_R * _N,), jnp.float32),
        ],
    )
    return fn(edge_index, edge_type)


def _tc_body(deg_ref, cnt_ref, x_ref, wrel_ref, w0_ref,
             wg_ref, bg_ref, ws_ref, bs_ref, lg_ref, ls_ref):
    deg = deg_ref[0] + 1.0
    dinv = lax.rsqrt(deg)
    d0 = dinv[:, 0:1]
    A = cnt_ref[0] * dinv * d0
    col0 = lax.broadcasted_iota(jnp.int32, (_R, _N), 1) == 0
    A = A + jnp.where(col0, d0 * d0, jnp.float32(0.0))

    xb = x_ref[0]
    y = jnp.dot(A, xb, preferred_element_type=jnp.float32)
    acc = jnp.dot(xb[0:1, :], w0_ref[...], preferred_element_type=jnp.float32)
    for r in range(_R):
        acc = acc + jnp.dot(y[r:r + 1, :], wrel_ref[r],
                            preferred_element_type=jnp.float32)
    x1 = jnp.where(acc >= 0, acc, 0.1 * acc)

    def head(w_ref, b_ref, n_real, n_pad):
        z = jnp.dot(x1, w_ref[...], preferred_element_type=jnp.float32) + b_ref[...]
        if n_pad > n_real:
            live = lax.broadcasted_iota(jnp.int32, (1, n_pad), 1) < n_real
            z = jnp.where(live, z, jnp.float32(-1e30))
        m = jnp.max(z, axis=1, keepdims=True)
        e = jnp.exp(z - m)
        return z - m - jnp.log(jnp.sum(e, axis=1, keepdims=True))

    lg_ref[0] = head(wg_ref, bg_ref, _NG, _NG)
    ls_ref[0] = head(ws_ref, bs_ref, _NS, _NS)


def _tc_call(deg, cnt, x, W_rel, W_0, W_g, b_g, W_s, b_s):
    grid = (_B,)
    full = lambda *shape: pl.BlockSpec(shape, lambda b: (0,) * len(shape))
    in_specs = [
        pl.BlockSpec((1, _R, _N), lambda b: (b, 0, 0)),
        pl.BlockSpec((1, _R, _N), lambda b: (b, 0, 0)),
        pl.BlockSpec((1, _N, _D), lambda b: (b, 0, 0)),
        full(_R, _D, _D),
        full(_D, _D),
        full(_D, _NG),
        full(1, _NG),
        full(_D, _NS),
        full(1, _NS),
    ]
    out_specs = [
        pl.BlockSpec((1, 1, _NG), lambda b: (b, 0, 0)),
        pl.BlockSpec((1, 1, _NS), lambda b: (b, 0, 0)),
    ]
    out_shape = [
        jax.ShapeDtypeStruct((_B, 1, _NG), jnp.float32),
        jax.ShapeDtypeStruct((_B, 1, _NS), jnp.float32),
    ]
    lg, ls = pl.pallas_call(
        _tc_body, grid=grid, in_specs=in_specs, out_specs=out_specs,
        out_shape=out_shape,
    )(deg, cnt, x, W_rel, W_0, W_g, b_g.reshape(1, _NG), W_s, b_s.reshape(1, _NS))
    return lg.reshape(_B, _NG), ls.reshape(_B, _NS)


def kernel(x, edge_index, edge_type, W_rel, W_0, W_g, b_g, W_s, b_s):
    deg_flat, cnt_flat = _sc_histograms(edge_index, edge_type)
    deg = deg_flat.reshape(_B, _R, _N)
    cnt = cnt_flat.reshape(_B, _R, _N)
    return _tc_call(deg, cnt, x, W_rel, W_0, W_g, b_g, W_s, b_s)


# speedup vs baseline: 1581.5187x; 1.6010x over previous
import jax
import jax.numpy as jnp
from jax import lax
from jax.experimental import pallas as pl
from jax.experimental.pallas import tpu as pltpu
from jax.experimental.pallas import tpu_sc as plsc

_B, _N, _E, _D, _R = 32, 512, 8192, 256, 8
_NG, _NS = 10000, 2000
_L = 16


def _sc_hist_body(ei_hbm, et_hbm, deg_hbm, cnt_hbm,
                  src_v, dst_v, et_v, histd_v, histc_v):
    c = lax.axis_index("c")
    s = lax.axis_index("s")
    w = s * 2 + c

    pltpu.sync_copy(ei_hbm.at[w, 0], src_v)
    pltpu.sync_copy(ei_hbm.at[w, 1], dst_v)
    pltpu.sync_copy(et_hbm.at[w], et_v)

    zeros = jnp.zeros((_L,), jnp.float32)

    def zero_body(i, carry):
        histd_v[pl.ds(i * _L, _L)] = zeros
        histc_v[pl.ds(i * _L, _L)] = zeros
        return carry

    lax.fori_loop(0, (_R * _N) // _L, zero_body, 0)

    ones = jnp.ones((_L,), jnp.float32)

    def edge_body(i, carry):
        off = i * _L
        sv = src_v[pl.ds(off, _L)]
        dv = dst_v[pl.ds(off, _L)]
        tv = et_v[pl.ds(off, _L)]
        base = tv * _N
        plsc.addupdate_scatter(histd_v, [base + dv], ones)
        plsc.addupdate_scatter(histc_v, [base + sv], ones, mask=dv == 0)
        return carry

    lax.fori_loop(0, _E // _L, edge_body, 0)

    pltpu.sync_copy(histd_v, deg_hbm.at[w])
    pltpu.sync_copy(histc_v, cnt_hbm.at[w])


def _sc_histograms(edge_index, edge_type):
    mesh = plsc.VectorSubcoreMesh(core_axis_name="c", subcore_axis_name="s")
    fn = pl.kernel(
        _sc_hist_body,
        out_type=[
            jax.ShapeDtypeStruct((_B, _R * _N), jnp.float32),
            jax.ShapeDtypeStruct((_B, _R * _N), jnp.float32),
        ],
        mesh=mesh,
        compiler_params=pltpu.CompilerParams(needs_layout_passes=False),
        scratch_types=[
            pltpu.VMEM((_E,), jnp.int32),
            pltpu.VMEM((_E,), jnp.int32),
            pltpu.VMEM((_E,), jnp.int32),
            pltpu.VMEM((_R * _N,), jnp.float32),
            pltpu.VMEM((_R * _N,), jnp.float32),
        ],
    )
    return fn(edge_index, edge_type)


def _tc_body(deg_ref, cnt_ref, x_ref, x0_ref, wrel_ref, w0_ref,
             wg_ref, bg_ref, ws_ref, bs_ref, lg_ref, ls_ref, y_s):
    b = pl.program_id(0)

    @pl.when(b < _B)
    def per_sample():
        deg = deg_ref[0] + 1.0
        dinv = lax.rsqrt(deg)
        d0 = dinv[:, 0:1]
        A = cnt_ref[0] * dinv * d0
        col0 = lax.broadcasted_iota(jnp.int32, (_R, _N), 1) == 0
        A = A + jnp.where(col0, d0 * d0, jnp.float32(0.0))
        xb = x_ref[0]
        y_s[pl.ds(b, 1)] = jnp.dot(
            A, xb, preferred_element_type=jnp.float32)[None]

    @pl.when(b == _B)
    def final():
        acc = jnp.dot(x0_ref[...], w0_ref[...],
                      preferred_element_type=jnp.float32)
        for r in range(_R):
            acc = acc + jnp.dot(y_s[:, r, :], wrel_ref[r],
                                preferred_element_type=jnp.float32)
        x1 = jnp.where(acc >= 0, acc, 0.1 * acc)

        def head(w_ref, b_ref):
            z = jnp.dot(x1, w_ref[...], preferred_element_type=jnp.float32) + b_ref[...]
            m = jnp.max(z, axis=1, keepdims=True)
            e = jnp.exp(z - m)
            return z - m - jnp.log(jnp.sum(e, axis=1, keepdims=True))

        lg_ref[...] = head(wg_ref, bg_ref)
        ls_ref[...] = head(ws_ref, bs_ref)


def _tc_call(deg, cnt, x, W_rel, W_0, W_g, b_g, W_s, b_s):
    grid = (_B + 1,)
    full = lambda *shape: pl.BlockSpec(shape, lambda b: (0,) * len(shape))
    clamp = lambda b: (jnp.minimum(b, _B - 1), 0, 0)
    in_specs = [
        pl.BlockSpec((1, _R, _N), clamp),
        pl.BlockSpec((1, _R, _N), clamp),
        pl.BlockSpec((1, _N, _D), clamp),
        full(_B, _D),
        full(_R, _D, _D),
        full(_D, _D),
        full(_D, _NG),
        full(1, _NG),
        full(_D, _NS),
        full(1, _NS),
    ]
    out_specs = [
        full(_B, _NG),
        full(_B, _NS),
    ]
    out_shape = [
        jax.ShapeDtypeStruct((_B, _NG), jnp.float32),
        jax.ShapeDtypeStruct((_B, _NS), jnp.float32),
    ]
    lg, ls = pl.pallas_call(
        _tc_body, grid=grid, in_specs=in_specs, out_specs=out_specs,
        out_shape=out_shape,
        scratch_shapes=[pltpu.VMEM((_B, _R, _D), jnp.float32)],
    )(deg, cnt, x, x[:, 0, :], W_rel, W_0, W_g,
      b_g.reshape(1, _NG), W_s, b_s.reshape(1, _NS))
    return lg, ls


def kernel(x, edge_index, edge_type, W_rel, W_0, W_g, b_g, W_s, b_s):
    deg_flat, cnt_flat = _sc_histograms(edge_index, edge_type)
    deg = deg_flat.reshape(_B, _R, _N)
    cnt = cnt_flat.reshape(_B, _R, _N)
    return _tc_call(deg, cnt, x, W_rel, W_0, W_g, b_g, W_s, b_s)


# speedup vs baseline: 1625.0043x; 1.0275x over previous
import jax
import jax.numpy as jnp
from jax import lax
from jax.experimental import pallas as pl
from jax.experimental.pallas import tpu as pltpu
from jax.experimental.pallas import tpu_sc as plsc

_B, _N, _E, _D, _R = 32, 512, 8192, 256, 8
_NG, _NS = 10000, 2000
_L = 16


def _sc_body(x2_hbm, ei_hbm, et_hbm, s_hbm, deg0_hbm,
             src_v, dst_v, et_v, histd_v, srcl_v, etl_v, s_v, rows_v, d0_v,
             sem):
    c = lax.axis_index("c")
    sx = lax.axis_index("s")
    w = sx * 2 + c

    pltpu.sync_copy(ei_hbm.at[w, 0], src_v)
    pltpu.sync_copy(ei_hbm.at[w, 1], dst_v)
    pltpu.sync_copy(et_hbm.at[w], et_v)

    zeros = jnp.zeros((_L,), jnp.float32)

    def zero_hist(i, cr):
        histd_v[pl.ds(i * _L, _L)] = zeros
        return cr

    lax.fori_loop(0, (_R * _N) // _L, zero_hist, 0)

    def zero_s(i, cr):
        s_v[pl.ds(i * _L, _L)] = zeros
        return cr

    lax.fori_loop(0, (_R * _D) // _L, zero_s, 0)

    ones = jnp.ones((_L,), jnp.float32)

    def edge_body(i, cnt):
        off = i * _L
        sv = src_v[pl.ds(off, _L)]
        dv = dst_v[pl.ds(off, _L)]
        tv = et_v[pl.ds(off, _L)]
        plsc.addupdate_scatter(histd_v, [tv * _N + dv], ones)
        m = dv == 0
        pref = plsc.cumsum(m.astype(jnp.int32))
        pos = cnt + pref - 1
        plsc.store_scatter(srcl_v, [pos], sv, mask=m)
        plsc.store_scatter(etl_v, [pos], tv, mask=m)
        return cnt + jnp.max(pref)

    nlist = lax.fori_loop(0, _E // _L, edge_body, jnp.int32(0))

    lanes = lax.iota(jnp.int32, _L)

    def chunk_body(k, cr):
        off = k * _L
        sl = srcl_v[pl.ds(off, _L)]
        tl = etl_v[pl.ds(off, _L)]
        act = (off + lanes) < nlist
        sl = jnp.where(act, sl, 0)
        tl = jnp.where(act, tl, 0)
        d = plsc.load_gather(histd_v, [tl * _N + sl]) + 1.0
        di = plsc.bitcast(d, jnp.int32)
        y = plsc.bitcast(jnp.int32(0x5F3759DF) - (di >> 1), jnp.float32)
        half = 0.5 * d
        y = y * (1.5 - half * y * y)
        y = y * (1.5 - half * y * y)
        y = y * (1.5 - half * y * y)
        wgt = jnp.where(act, y, jnp.float32(0.0))
        gidx = jnp.where(act, w * _N + sl, 0)
        pltpu.async_copy(x2_hbm.at[gidx], rows_v, sem).wait()

        def col_body(cc, cr2):
            colv = plsc.load_gather(rows_v, [lanes, jnp.broadcast_to(cc, (_L,))])
            plsc.addupdate_scatter(s_v, [tl * _D + cc], wgt * colv, mask=act)
            return cr2

        lax.fori_loop(0, _D, col_body, 0)
        return cr

    nchunk = (nlist + _L - 1) // _L
    lax.fori_loop(0, nchunk, chunk_body, 0)

    d0_v[...] = plsc.load_gather(histd_v, [jnp.minimum(lanes * _N, _R * _N - 1)])
    pltpu.sync_copy(d0_v, deg0_hbm.at[w])
    pltpu.sync_copy(s_v, s_hbm.at[w])


def _sc_gather_sums(x, edge_index, edge_type):
    x2 = x.reshape(_B * _N, _D)
    mesh = plsc.VectorSubcoreMesh(core_axis_name="c", subcore_axis_name="s")
    fn = pl.kernel(
        _sc_body,
        out_type=[
            jax.ShapeDtypeStruct((_B, _R * _D), jnp.float32),
            jax.ShapeDtypeStruct((_B, _L), jnp.float32),
        ],
        mesh=mesh,
        compiler_params=pltpu.CompilerParams(needs_layout_passes=False),
        scratch_types=[
            pltpu.VMEM((_E,), jnp.int32),
            pltpu.VMEM((_E,), jnp.int32),
            pltpu.VMEM((_E,), jnp.int32),
            pltpu.VMEM((_R * _N,), jnp.float32),
            pltpu.VMEM((_E,), jnp.int32),
            pltpu.VMEM((_E,), jnp.int32),
            pltpu.VMEM((_R * _D,), jnp.float32),
            pltpu.VMEM((_L, _D), jnp.float32),
            pltpu.VMEM((_L,), jnp.float32),
            pltpu.SemaphoreType.DMA,
        ],
    )
    return fn(x2, edge_index, edge_type)


def _tc_body(s_ref, d0_ref, x0_ref, wrel_ref, w0_ref,
             wg_ref, bg_ref, ws_ref, bs_ref, lg_ref, ls_ref):
    dinv0 = lax.rsqrt(d0_ref[...] + 1.0)
    x0 = x0_ref[...]
    acc = jnp.dot(x0, w0_ref[...], preferred_element_type=jnp.float32)
    for r in range(_R):
        dr = dinv0[:, r:r + 1]
        p = s_ref[:, r, :] * dr + x0 * (dr * dr)
        acc = acc + jnp.dot(p, wrel_ref[r], preferred_element_type=jnp.float32)
    x1 = jnp.where(acc >= 0, acc, 0.1 * acc)

    def head(w_ref, b_ref):
        z = jnp.dot(x1, w_ref[...], preferred_element_type=jnp.float32) + b_ref[...]
        m = jnp.max(z, axis=1, keepdims=True)
        e = jnp.exp(z - m)
        return z - m - jnp.log(jnp.sum(e, axis=1, keepdims=True))

    lg_ref[...] = head(wg_ref, bg_ref)
    ls_ref[...] = head(ws_ref, bs_ref)


def _tc_call(s, deg0, x0, W_rel, W_0, W_g, b_g, W_s, b_s):
    out_shape = [
        jax.ShapeDtypeStruct((_B, _NG), jnp.float32),
        jax.ShapeDtypeStruct((_B, _NS), jnp.float32),
    ]
    return pl.pallas_call(_tc_body, out_shape=out_shape)(
        s, deg0, x0, W_rel, W_0, W_g,
        b_g.reshape(1, _NG), W_s, b_s.reshape(1, _NS))


def kernel(x, edge_index, edge_type, W_rel, W_0, W_g, b_g, W_s, b_s):
    s_flat, deg0 = _sc_gather_sums(x, edge_index, edge_type)
    s = s_flat.reshape(_B, _R, _D)
    lg, ls = _tc_call(s, deg0, x[:, 0, :], W_rel, W_0, W_g, b_g, W_s, b_s)
    return lg, ls
